# baseline (speedup 1.0000x reference)
import functools

import jax
import jax.numpy as jnp
from jax import lax
from jax.experimental import pallas as pl
from jax.experimental.pallas import tpu as pltpu
from jax.experimental.pallas import tpu_sc as plsc

B = 16384
D = 64
NC = 2
NS = 16
NW = NC * NS
CHUNK = 128
ROWS_PER_W = B // NW
CHUNKS_PER_W = ROWS_PER_W // CHUNK


def _sc_body(ct_hbm, g_hbm, pe_hbm, po_hbm,
             wc0, wc1, wg0, wg1,
             out0, out1,
             ct_v, g_v, pe_v, po_v, rows_a, rows_b, sem_a, sem_b):
    wid = lax.axis_index("s") * NC + lax.axis_index("c")
    r0 = wid * CHUNKS_PER_W

    pltpu.sync_copy(ct_hbm.at[pl.ds(r0, CHUNKS_PER_W)], ct_v)
    pltpu.sync_copy(g_hbm.at[pl.ds(r0, CHUNKS_PER_W)], g_v)
    pltpu.sync_copy(pe_hbm.at[pl.ds(r0, CHUNKS_PER_W)], pe_v)
    pltpu.sync_copy(po_hbm.at[pl.ds(r0, CHUNKS_PER_W)], po_v)

    work = (
        (wc0, ct_v, pe_v, out0),
        (wg0, g_v, po_v, out0),
        (wc1, ct_v, pe_v, out1),
        (wg1, g_v, po_v, out1),
    )
    bufs = (rows_a, rows_b)
    sems = (sem_a, sem_b)
    flat = [(tab, idx_v.at[j], pos_v.at[j], out)
            for (tab, idx_v, pos_v, out) in work
            for j in range(CHUNKS_PER_W)]
    n = len(flat)
    gathers = [None] * n
    scatters = [None] * n
    for k, (tab, idx, pos, out) in enumerate(flat):
        p = k % 2
        if k >= 2:
            scatters[k - 2].wait()
        gathers[k] = pltpu.async_copy(tab.at[idx], bufs[p], sems[p])
        gathers[k].wait()
        scatters[k] = pltpu.async_copy(bufs[p], out.at[pos], sems[p])
    scatters[n - 2].wait()
    scatters[n - 1].wait()


@functools.partial(jax.jit, static_argnums=())
def kernel(celltype, gene, W_celltype_0, W_celltype_1, W_gene_0, W_gene_1):
    nrows = B // CHUNK
    ct2 = celltype.astype(jnp.int32).reshape(nrows, CHUNK)
    g2 = gene.astype(jnp.int32).reshape(nrows, CHUNK)
    pos_even = (jnp.arange(B, dtype=jnp.int32) * 2).reshape(nrows, CHUNK)
    pos_odd = pos_even + 1

    mesh = plsc.VectorSubcoreMesh(core_axis_name="c", subcore_axis_name="s")
    out0, out1 = pl.kernel(
        _sc_body,
        out_type=(
            jax.ShapeDtypeStruct((2 * B, D), jnp.float32),
            jax.ShapeDtypeStruct((2 * B, D), jnp.float32),
        ),
        mesh=mesh,
        scratch_types=[
            pltpu.VMEM((CHUNKS_PER_W, CHUNK), jnp.int32),
            pltpu.VMEM((CHUNKS_PER_W, CHUNK), jnp.int32),
            pltpu.VMEM((CHUNKS_PER_W, CHUNK), jnp.int32),
            pltpu.VMEM((CHUNKS_PER_W, CHUNK), jnp.int32),
            pltpu.VMEM((CHUNK, D), jnp.float32),
            pltpu.VMEM((CHUNK, D), jnp.float32),
            pltpu.SemaphoreType.DMA,
            pltpu.SemaphoreType.DMA,
        ],
        compiler_params=pltpu.CompilerParams(use_tc_tiling_on_sc=False),
        name="embedding_dict_sc",
    )(ct2, g2, pos_even, pos_odd, W_celltype_0, W_celltype_1, W_gene_0, W_gene_1)

    return (out0.reshape(B, 2, D), out1.reshape(B, 2, D))


# speedup vs baseline: 1.0241x; 1.0241x over previous
import jax
import jax.numpy as jnp
from jax import lax
from jax.experimental import pallas as pl
from jax.experimental.pallas import tpu as pltpu
from jax.experimental.pallas import tpu_sc as plsc

B = 16384
D = 64
NC = 2
NS = 16
NW = NC * NS
CHUNK = 128
ROWS_PER_W = B // NW
NCHUNK = ROWS_PER_W // CHUNK
NBUF = 12
NUNITS = 4 * NCHUNK


def _sc_body(ct_hbm, g_hbm, pe_hbm, po_hbm,
             wc0, wc1, wg0, wg1,
             out0, out1,
             ct_v, g_v, pe_v, po_v, rows, gsem, ssem):
    wid = lax.axis_index("s") * NC + lax.axis_index("c")
    r0 = wid * NCHUNK

    pltpu.sync_copy(ct_hbm.at[pl.ds(r0, NCHUNK)], ct_v)
    pltpu.sync_copy(g_hbm.at[pl.ds(r0, NCHUNK)], g_v)
    pltpu.sync_copy(pe_hbm.at[pl.ds(r0, NCHUNK)], pe_v)
    pltpu.sync_copy(po_hbm.at[pl.ds(r0, NCHUNK)], po_v)

    units = [(tab, idx_v.at[j], pos_v.at[j], out)
             for (tab, idx_v, pos_v, out) in (
                 (wc0, ct_v, pe_v, out0),
                 (wg0, g_v, po_v, out0),
                 (wc1, ct_v, pe_v, out1),
                 (wg1, g_v, po_v, out1),
             )
             for j in range(NCHUNK)]

    def gather(k):
        tab, idx, _, _ = units[k]
        b = k % NBUF
        return pltpu.async_copy(tab.at[idx], rows.at[b], gsem.at[b])

    def scatter(k):
        _, _, pos, out = units[k]
        b = k % NBUF
        return pltpu.async_copy(rows.at[b], out.at[pos], ssem.at[b])

    gathers = [None] * NUNITS
    scatters = [None] * NUNITS
    for k in range(min(NBUF, NUNITS)):
        gathers[k] = gather(k)
    for k in range(NUNITS):
        gathers[k].wait()
        scatters[k] = scatter(k)
        if k + NBUF < NUNITS:
            scatters[k].wait()
            gathers[k + NBUF] = gather(k + NBUF)
    for k in range(max(0, NUNITS - NBUF), NUNITS):
        scatters[k].wait()


def kernel(celltype, gene, W_celltype_0, W_celltype_1, W_gene_0, W_gene_1):
    nrows = B // CHUNK
    ct2 = celltype.astype(jnp.int32).reshape(nrows, CHUNK)
    g2 = gene.astype(jnp.int32).reshape(nrows, CHUNK)
    pos_even = (jnp.arange(B, dtype=jnp.int32) * 2).reshape(nrows, CHUNK)
    pos_odd = pos_even + 1

    mesh = plsc.VectorSubcoreMesh(core_axis_name="c", subcore_axis_name="s")
    out0, out1 = pl.kernel(
        _sc_body,
        out_type=(
            jax.ShapeDtypeStruct((2 * B, D), jnp.float32),
            jax.ShapeDtypeStruct((2 * B, D), jnp.float32),
        ),
        mesh=mesh,
        scratch_types=[
            pltpu.VMEM((NCHUNK, CHUNK), jnp.int32),
            pltpu.VMEM((NCHUNK, CHUNK), jnp.int32),
            pltpu.VMEM((NCHUNK, CHUNK), jnp.int32),
            pltpu.VMEM((NCHUNK, CHUNK), jnp.int32),
            pltpu.VMEM((NBUF, CHUNK, D), jnp.float32),
            pltpu.SemaphoreType.DMA((NBUF,)),
            pltpu.SemaphoreType.DMA((NBUF,)),
        ],
        compiler_params=pltpu.CompilerParams(use_tc_tiling_on_sc=False),
        name="embedding_dict_sc",
    )(ct2, g2, pos_even, pos_odd, W_celltype_0, W_celltype_1, W_gene_0, W_gene_1)

    return (out0.reshape(B, 2, D), out1.reshape(B, 2, D))


# speedup vs baseline: 1.3114x; 1.2805x over previous
import jax
import jax.numpy as jnp
from jax import lax
from jax.experimental import pallas as pl
from jax.experimental.pallas import tpu as pltpu
from jax.experimental.pallas import tpu_sc as plsc

B = 16384
D = 64
NC = 2
NS = 16
NW = NC * NS
CHUNK = 128
ROWS_PER_W = B // NW
NCHUNK = ROWS_PER_W // CHUNK
NBUF = 3
NUNITS = 2 * NCHUNK
LANES = 16


def _interleave(cbuf, gbuf, ibuf):
    def row(i, carry):
        for c in range(D // LANES):
            ibuf[i, pl.ds(c * LANES, LANES)] = cbuf[i, pl.ds(c * LANES, LANES)]
            ibuf[i, pl.ds(D + c * LANES, LANES)] = gbuf[i, pl.ds(c * LANES, LANES)]
        return carry
    lax.fori_loop(0, CHUNK, row, 0)


def _sc_body(ct_hbm, g_hbm,
             wc0, wc1, wg0, wg1,
             out0, out1,
             ct_v, g_v, cbuf, gbuf, ibuf, gsem, ssem):
    wid = lax.axis_index("s") * NC + lax.axis_index("c")
    r0 = wid * NCHUNK
    base = wid * ROWS_PER_W

    pltpu.sync_copy(ct_hbm.at[pl.ds(r0, NCHUNK)], ct_v)
    pltpu.sync_copy(g_hbm.at[pl.ds(r0, NCHUNK)], g_v)

    units = [(tc, tg, j, out)
             for (tc, tg, out) in ((wc0, wg0, out0), (wc1, wg1, out1))
             for j in range(NCHUNK)]

    def gathers_for(u):
        tc, tg, j, _ = units[u]
        s = u % NBUF
        return (pltpu.async_copy(tc.at[ct_v.at[j]], cbuf.at[s], gsem.at[s]),
                pltpu.async_copy(tg.at[g_v.at[j]], gbuf.at[s], gsem.at[s]))

    gth = [None] * NUNITS
    sto = [None] * NUNITS
    for u in range(NBUF):
        gth[u] = gathers_for(u)
    for u in range(NUNITS):
        s = u % NBUF
        if u >= NBUF:
            sto[u - NBUF].wait()
        gth[u][0].wait()
        gth[u][1].wait()
        _interleave(cbuf.at[s], gbuf.at[s], ibuf.at[s])
        if u + NBUF < NUNITS:
            gth[u + NBUF] = gathers_for(u + NBUF)
        _, _, j, out = units[u]
        sto[u] = pltpu.async_copy(
            ibuf.at[s], out.at[pl.ds(base + j * CHUNK, CHUNK)], ssem.at[s])
    for u in range(NUNITS - NBUF, NUNITS):
        sto[u].wait()


def kernel(celltype, gene, W_celltype_0, W_celltype_1, W_gene_0, W_gene_1):
    nrows = B // CHUNK
    ct2 = celltype.astype(jnp.int32).reshape(nrows, CHUNK)
    g2 = gene.astype(jnp.int32).reshape(nrows, CHUNK)

    mesh = plsc.VectorSubcoreMesh(core_axis_name="c", subcore_axis_name="s")
    out0, out1 = pl.kernel(
        _sc_body,
        out_type=(
            jax.ShapeDtypeStruct((B, 2 * D), jnp.float32),
            jax.ShapeDtypeStruct((B, 2 * D), jnp.float32),
        ),
        mesh=mesh,
        scratch_types=[
            pltpu.VMEM((NCHUNK, CHUNK), jnp.int32),
            pltpu.VMEM((NCHUNK, CHUNK), jnp.int32),
            pltpu.VMEM((NBUF, CHUNK, D), jnp.float32),
            pltpu.VMEM((NBUF, CHUNK, D), jnp.float32),
            pltpu.VMEM((NBUF, CHUNK, 2 * D), jnp.float32),
            pltpu.SemaphoreType.DMA((NBUF,)),
            pltpu.SemaphoreType.DMA((NBUF,)),
        ],
        compiler_params=pltpu.CompilerParams(use_tc_tiling_on_sc=False),
        name="embedding_dict_sc",
    )(ct2, g2, W_celltype_0, W_celltype_1, W_gene_0, W_gene_1)

    return (out0.reshape(B, 2, D), out1.reshape(B, 2, D))


# speedup vs baseline: 1.4609x; 1.1140x over previous
import jax
import jax.numpy as jnp
from jax import lax
from jax.experimental import pallas as pl
from jax.experimental.pallas import tpu as pltpu
from jax.experimental.pallas import tpu_sc as plsc

B = 16384
D = 64
NC = 2
NS = 16
NW = NC * NS
CHUNK = 128
ROWS_PER_W = B // NW
NCHUNK = ROWS_PER_W // CHUNK
NUNITS = 2 * NCHUNK
NBUF = 7


def _sc_body(ct_hbm, g_hbm,
             wc0, wc1, wg0, wg1,
             out0, out1,
             ct_v, g_v, cbuf, gbuf, gsem, ssem):
    wid = lax.axis_index("s") * NC + lax.axis_index("c")
    r0 = wid * NCHUNK
    base = wid * ROWS_PER_W

    pltpu.sync_copy(ct_hbm.at[pl.ds(r0, NCHUNK)], ct_v)
    pltpu.sync_copy(g_hbm.at[pl.ds(r0, NCHUNK)], g_v)

    units = [(tc, tg, j, out)
             for (tc, tg, out) in ((wc0, wg0, out0), (wc1, wg1, out1))
             for j in range(NCHUNK)]

    def fire_gathers(u):
        tc, tg, j, _ = units[u]
        s = u % NBUF
        return (pltpu.async_copy(tc.at[ct_v.at[j]], cbuf.at[s], gsem.at[s]),
                pltpu.async_copy(tg.at[g_v.at[j]], gbuf.at[s], gsem.at[s]))

    gth = [None] * NUNITS
    sto = [None] * NUNITS
    for u in range(min(NBUF, NUNITS)):
        gth[u] = fire_gathers(u)
    for u in range(NUNITS):
        s = u % NBUF
        tc, tg, j, out = units[u]
        gth[u][0].wait()
        gth[u][1].wait()
        rows = out.at[pl.ds(base + j * CHUNK, CHUNK)]
        sto[u] = (pltpu.async_copy(cbuf.at[s], rows.at[:, pl.ds(0, D)], ssem.at[s]),
                  pltpu.async_copy(gbuf.at[s], rows.at[:, pl.ds(D, D)], ssem.at[s]))
        if u + NBUF < NUNITS:
            sto[u][0].wait()
            sto[u][1].wait()
            gth[u + NBUF] = fire_gathers(u + NBUF)
    for u in range(max(0, NUNITS - NBUF), NUNITS):
        sto[u][0].wait()
        sto[u][1].wait()


def kernel(celltype, gene, W_celltype_0, W_celltype_1, W_gene_0, W_gene_1):
    nrows = B // CHUNK
    ct2 = celltype.astype(jnp.int32).reshape(nrows, CHUNK)
    g2 = gene.astype(jnp.int32).reshape(nrows, CHUNK)

    mesh = plsc.VectorSubcoreMesh(core_axis_name="c", subcore_axis_name="s")
    out0, out1 = pl.kernel(
        _sc_body,
        out_type=(
            jax.ShapeDtypeStruct((B, 2 * D), jnp.float32),
            jax.ShapeDtypeStruct((B, 2 * D), jnp.float32),
        ),
        mesh=mesh,
        scratch_types=[
            pltpu.VMEM((NCHUNK, CHUNK), jnp.int32),
            pltpu.VMEM((NCHUNK, CHUNK), jnp.int32),
            pltpu.VMEM((NBUF, CHUNK, D), jnp.float32),
            pltpu.VMEM((NBUF, CHUNK, D), jnp.float32),
            pltpu.SemaphoreType.DMA((NBUF,)),
            pltpu.SemaphoreType.DMA((NBUF,)),
        ],
        compiler_params=pltpu.CompilerParams(use_tc_tiling_on_sc=False),
        name="embedding_dict_sc",
    )(ct2, g2, W_celltype_0, W_celltype_1, W_gene_0, W_gene_1)

    return (out0.reshape(B, 2, D), out1.reshape(B, 2, D))


# speedup vs baseline: 1.6148x; 1.1054x over previous
import jax
import jax.numpy as jnp
from jax import lax
from jax.experimental import pallas as pl
from jax.experimental.pallas import tpu as pltpu
from jax.experimental.pallas import tpu_sc as plsc

B = 16384
D = 64
NC = 2
NS = 16
NW = NC * NS
CHUNK = 128
ROWS_PER_W = B // NW
NCHUNK = ROWS_PER_W // CHUNK
NUNITS = 2 * NCHUNK
NBUF = 7


def _sc_body(ct_hbm, g_hbm, wct, wg,
             out0, out1,
             ct_v, g_v, buf, gsem, ssem):
    wid = lax.axis_index("s") * NC + lax.axis_index("c")
    r0 = wid * NCHUNK
    base = wid * ROWS_PER_W

    pltpu.sync_copy(ct_hbm.at[pl.ds(r0, NCHUNK)], ct_v)
    pltpu.sync_copy(g_hbm.at[pl.ds(r0, NCHUNK)], g_v)

    units = [(tab, idx_v, half, j)
             for (tab, idx_v, half) in ((wct, ct_v, 0), (wg, g_v, D))
             for j in range(NCHUNK)]

    def fire_gather(u):
        tab, idx_v, _, j = units[u]
        return pltpu.async_copy(tab.at[idx_v.at[j]], buf.at[u % NBUF],
                                gsem.at[u % NBUF])

    gth = [None] * NUNITS
    sto = [None] * NUNITS
    for u in range(min(NBUF, NUNITS)):
        gth[u] = fire_gather(u)
    for u in range(NUNITS):
        s = u % NBUF
        _, _, half, j = units[u]
        gth[u].wait()
        rows = pl.ds(base + j * CHUNK, CHUNK)
        sto[u] = (
            pltpu.async_copy(buf.at[s, :, pl.ds(0, D)],
                             out0.at[rows, pl.ds(half, D)], ssem.at[s]),
            pltpu.async_copy(buf.at[s, :, pl.ds(D, D)],
                             out1.at[rows, pl.ds(half, D)], ssem.at[s]),
        )
        if u + NBUF < NUNITS:
            sto[u][0].wait()
            sto[u][1].wait()
            gth[u + NBUF] = fire_gather(u + NBUF)
    for u in range(max(0, NUNITS - NBUF), NUNITS):
        sto[u][0].wait()
        sto[u][1].wait()


def kernel(celltype, gene, W_celltype_0, W_celltype_1, W_gene_0, W_gene_1):
    nrows = B // CHUNK
    ct2 = celltype.astype(jnp.int32).reshape(nrows, CHUNK)
    g2 = gene.astype(jnp.int32).reshape(nrows, CHUNK)
    wct = jnp.concatenate([W_celltype_0, W_celltype_1], axis=1)
    wg = jnp.concatenate([W_gene_0, W_gene_1], axis=1)

    mesh = plsc.VectorSubcoreMesh(core_axis_name="c", subcore_axis_name="s")
    out0, out1 = pl.kernel(
        _sc_body,
        out_type=(
            jax.ShapeDtypeStruct((B, 2 * D), jnp.float32),
            jax.ShapeDtypeStruct((B, 2 * D), jnp.float32),
        ),
        mesh=mesh,
        scratch_types=[
            pltpu.VMEM((NCHUNK, CHUNK), jnp.int32),
            pltpu.VMEM((NCHUNK, CHUNK), jnp.int32),
            pltpu.VMEM((NBUF, CHUNK, 2 * D), jnp.float32),
            pltpu.SemaphoreType.DMA((NBUF,)),
            pltpu.SemaphoreType.DMA((NBUF,)),
        ],
        compiler_params=pltpu.CompilerParams(use_tc_tiling_on_sc=False),
        name="embedding_dict_sc",
    )(ct2, g2, wct, wg)

    return (out0.reshape(B, 2, D), out1.reshape(B, 2, D))


# speedup vs baseline: 1.6912x; 1.0473x over previous
import jax
import jax.numpy as jnp
from jax import lax
from jax.experimental import pallas as pl
from jax.experimental.pallas import tpu as pltpu
from jax.experimental.pallas import tpu_sc as plsc

B = 16384
D = 64
NC = 2
NS = 16
NW = NC * NS
CHUNK = 128
ROWS_PER_W = B // NW
NCHUNK = ROWS_PER_W // CHUNK
NBUF = 2
LANES = 16


def _swap_halves(cbuf, gbuf):
    def row(i, carry):
        for c in range(D // LANES):
            hi = pl.ds(D + c * LANES, LANES)
            lo = pl.ds(c * LANES, LANES)
            t = cbuf[i, hi]
            cbuf[i, hi] = gbuf[i, lo]
            gbuf[i, lo] = t
        return carry
    lax.fori_loop(0, CHUNK, row, 0)


def _gather_body(ct_hbm, g_hbm, wct, wg,
                 out0, out1,
                 cti_v, gi_v, cbuf, gbuf, gsem, ssem):
    wid = lax.axis_index("s") * NC + lax.axis_index("c")
    grp = (wid // 8) * 8
    row = wid % 8
    base = wid * ROWS_PER_W

    pltpu.sync_copy(ct_hbm.at[pl.ds(grp, 8)], cti_v)
    pltpu.sync_copy(g_hbm.at[pl.ds(grp, 8)], gi_v)

    def fire_gathers(j):
        s = j % NBUF
        cols = pl.ds(j * CHUNK, CHUNK)
        return (pltpu.async_copy(wct.at[cti_v.at[row, cols]], cbuf.at[s], gsem.at[s]),
                pltpu.async_copy(wg.at[gi_v.at[row, cols]], gbuf.at[s], gsem.at[s]))

    gth = [None] * NCHUNK
    sto = [None] * NCHUNK
    for j in range(NBUF):
        gth[j] = fire_gathers(j)
    for j in range(NCHUNK):
        s = j % NBUF
        gth[j][0].wait()
        gth[j][1].wait()
        _swap_halves(cbuf.at[s], gbuf.at[s])
        rows = pl.ds(base + j * CHUNK, CHUNK)
        sto[j] = (pltpu.async_copy(cbuf.at[s], out0.at[rows], ssem.at[s]),
                  pltpu.async_copy(gbuf.at[s], out1.at[rows], ssem.at[s]))
        if j + NBUF < NCHUNK:
            sto[j][0].wait()
            sto[j][1].wait()
            gth[j + NBUF] = fire_gathers(j + NBUF)
    for j in range(max(0, NCHUNK - NBUF), NCHUNK):
        sto[j][0].wait()
        sto[j][1].wait()


def kernel(celltype, gene, W_celltype_0, W_celltype_1, W_gene_0, W_gene_1):
    ct2 = celltype.astype(jnp.int32).reshape(NW, ROWS_PER_W)
    g2 = gene.astype(jnp.int32).reshape(NW, ROWS_PER_W)
    wct = jnp.concatenate([W_celltype_0, W_celltype_1], axis=1)
    wg = jnp.concatenate([W_gene_0, W_gene_1], axis=1)

    mesh = plsc.VectorSubcoreMesh(core_axis_name="c", subcore_axis_name="s")
    out0, out1 = pl.kernel(
        _gather_body,
        out_type=(
            jax.ShapeDtypeStruct((B, 2 * D), jnp.float32),
            jax.ShapeDtypeStruct((B, 2 * D), jnp.float32),
        ),
        mesh=mesh,
        scratch_types=[
            pltpu.VMEM((8, ROWS_PER_W), jnp.int32),
            pltpu.VMEM((8, ROWS_PER_W), jnp.int32),
            pltpu.VMEM((NBUF, CHUNK, 2 * D), jnp.float32),
            pltpu.VMEM((NBUF, CHUNK, 2 * D), jnp.float32),
            pltpu.SemaphoreType.DMA((NBUF,)),
            pltpu.SemaphoreType.DMA((NBUF,)),
        ],
        compiler_params=pltpu.CompilerParams(use_tc_tiling_on_sc=True),
        name="embedding_dict_sc",
    )(ct2, g2, wct, wg)

    return (out0.reshape(B, 2, D), out1.reshape(B, 2, D))
